# baseline (speedup 1.0000x reference)
import functools

import jax
import jax.numpy as jnp
from jax import lax
from jax.experimental import pallas as pl
from jax.experimental.pallas import tpu as pltpu
from jax.experimental.pallas import tpu_sc as plsc

N_U = 5000
N_I = 5000
NPAD = 5120
D = 128
E = 320000
TEMP = 0.2
LAMBDA_1 = 0.2
LAMBDA_2 = 1e-07
LAMBDA_3 = 1e-05
MLP_COF = 1.0
B = 1024

NC = 2
NS = 16
NW = NC * NS
EPW = E // NW
C = 80
NCH = EPW // C
RPT = NPAD // NS
G = 25

_f32 = jnp.float32
_i32 = jnp.int32


def _spmm_body(scaled, ti, tu, src3, dst3, w2, outu, outi,
               src_v, dst_v, w_v, rows_u, rows_i, zbuf, acc_u, acc_i,
               sem_u, sem_i):
    c = lax.axis_index("c")
    s = lax.axis_index("s")
    wid = s * NC + c

    zero = jnp.zeros((16,), _f32)
    for r in range(8):
        for k in range(8):
            zbuf[r, pl.ds(k * 16, 16)] = zero
    for b in range(RPT // 8):
        pltpu.sync_copy(zbuf, acc_u.at[pl.ds(s * RPT + b * 8, 8)])
        pltpu.sync_copy(zbuf, acc_i.at[pl.ds(s * RPT + b * 8, 8)])
    plsc.subcore_barrier()

    for g in range(NCH // G):
        pltpu.sync_copy(src3.at[wid, g], src_v)
        pltpu.sync_copy(dst3.at[wid, g], dst_v)
        if scaled:
            pltpu.sync_copy(w2.at[pl.ds(wid * EPW + g * G * C, G * C)], w_v)

        def chunk(j, carry):
            pltpu.async_copy(ti.at[dst_v.at[j]], rows_u, sem_u).wait()
            pltpu.async_copy(tu.at[src_v.at[j]], rows_i, sem_i).wait()
            if scaled:
                def edge(e, cc):
                    wsp = plsc.load_gather(w_v, [jnp.full((16,), j * C + e, _i32)])
                    for k in range(8):
                        rows_u[e, pl.ds(k * 16, 16)] = rows_u[e, pl.ds(k * 16, 16)] * wsp
                        rows_i[e, pl.ds(k * 16, 16)] = rows_i[e, pl.ds(k * 16, 16)] * wsp
                    return cc
                lax.fori_loop(0, C, edge, 0)
            pltpu.sync_copy(rows_u, acc_u.at[src_v.at[j]], add=True)
            pltpu.sync_copy(rows_i, acc_i.at[dst_v.at[j]], add=True)
            return carry

        lax.fori_loop(0, G, chunk, 0)
    plsc.subcore_barrier()

    pltpu.sync_copy(acc_u.at[pl.ds(s * RPT, RPT)], outu.at[c, pl.ds(s * RPT, RPT)])
    pltpu.sync_copy(acc_i.at[pl.ds(s * RPT, RPT)], outi.at[c, pl.ds(s * RPT, RPT)])


def _make_spmm(scaled):
    mesh = plsc.VectorSubcoreMesh(core_axis_name="c", subcore_axis_name="s")
    scratch = [
        pltpu.VMEM((G, C), _i32),
        pltpu.VMEM((G, C), _i32),
        pltpu.VMEM((G * C,), _f32),
        pltpu.VMEM((C, D), _f32),
        pltpu.VMEM((C, D), _f32),
        pltpu.VMEM((8, D), _f32),
        pltpu.VMEM_SHARED((NPAD, D), _f32),
        pltpu.VMEM_SHARED((NPAD, D), _f32),
        pltpu.SemaphoreType.DMA,
        pltpu.SemaphoreType.DMA,
    ]
    return pl.kernel(
        functools.partial(_spmm_body, scaled),
        out_type=(jax.ShapeDtypeStruct((NC, NPAD, D), _f32),
                  jax.ShapeDtypeStruct((NC, NPAD, D), _f32)),
        mesh=mesh,
        scratch_types=scratch,
        compiler_params=pltpu.CompilerParams(needs_layout_passes=False),
    )


_spmm_plain = _make_spmm(False)
_spmm_scaled = _make_spmm(True)


def _views_body(eu_t, ei_t, hu_t, hi_t, bw_t, pu_t, pi_t, src3, dst3,
                dgcn_o, dmlp_o, att_o,
                src_v, dst_v, eu_r, ei_r, hu_r, hi_r, bw_v, pu_v, pi_v,
                dg_v, dm_v, at_v, sem_a, sem_b, sem_c, sem_d):
    c = lax.axis_index("c")
    s = lax.axis_index("s")
    wid = s * NC + c

    pltpu.sync_copy(bw_t, bw_v)
    pltpu.sync_copy(pu_t, pu_v)
    pltpu.sync_copy(pi_t, pi_v)

    for g in range(NCH // G):
        pltpu.sync_copy(src3.at[wid, g], src_v)
        pltpu.sync_copy(dst3.at[wid, g], dst_v)

        def chunk(j, carry):
            pltpu.async_copy(eu_t.at[src_v.at[j]], eu_r, sem_a).wait()
            pltpu.async_copy(ei_t.at[dst_v.at[j]], ei_r, sem_b).wait()
            pltpu.async_copy(hu_t.at[src_v.at[j]], hu_r, sem_c).wait()
            pltpu.async_copy(hi_t.at[dst_v.at[j]], hi_r, sem_d).wait()

            def edge(e, cc):
                accg = jnp.zeros((16,), _f32)
                accm = jnp.zeros((16,), _f32)
                for k in range(8):
                    sl = pl.ds(k * 16, 16)
                    accg = accg + eu_r[e, sl] * ei_r[e, sl]
                    h = jnp.maximum(hu_r[e, sl] + hi_r[e, sl] + bw_v[0, sl], 0.0)
                    accm = accm + h * bw_v[1, sl]
                lane0 = lax.iota(_i32, 16) == 0
                eidx = jnp.full((16,), j * C + e, _i32)
                plsc.store_scatter(dg_v, [eidx],
                                   jnp.full((16,), jnp.sum(accg, axis=0), _f32),
                                   mask=lane0)
                plsc.store_scatter(dm_v, [eidx],
                                   jnp.full((16,), jnp.sum(accm, axis=0), _f32),
                                   mask=lane0)
                return cc
            lax.fori_loop(0, C, edge, 0)

            def att16(t, cc):
                si = src_v[j, pl.ds(t * 16, 16)]
                di = dst_v[j, pl.ds(t * 16, 16)]
                a = plsc.load_gather(pu_v, [si]) + plsc.load_gather(pi_v, [di])
                at_v[pl.ds(j * C + t * 16, 16)] = a
                return cc
            lax.fori_loop(0, C // 16, att16, 0)
            return carry

        lax.fori_loop(0, G, chunk, 0)
        base = wid * EPW + g * G * C
        pltpu.sync_copy(dg_v, dgcn_o.at[pl.ds(base, G * C)])
        pltpu.sync_copy(dm_v, dmlp_o.at[pl.ds(base, G * C)])
        pltpu.sync_copy(at_v, att_o.at[pl.ds(base, G * C)])


def _make_views():
    mesh = plsc.VectorSubcoreMesh(core_axis_name="c", subcore_axis_name="s")
    scratch = [
        pltpu.VMEM((G, C), _i32),
        pltpu.VMEM((G, C), _i32),
        pltpu.VMEM((C, D), _f32),
        pltpu.VMEM((C, D), _f32),
        pltpu.VMEM((C, D), _f32),
        pltpu.VMEM((C, D), _f32),
        pltpu.VMEM((2, D), _f32),
        pltpu.VMEM((N_U,), _f32),
        pltpu.VMEM((N_I,), _f32),
        pltpu.VMEM((G * C,), _f32),
        pltpu.VMEM((G * C,), _f32),
        pltpu.VMEM((G * C,), _f32),
        pltpu.SemaphoreType.DMA,
        pltpu.SemaphoreType.DMA,
        pltpu.SemaphoreType.DMA,
        pltpu.SemaphoreType.DMA,
    ]
    return pl.kernel(
        _views_body,
        out_type=(jax.ShapeDtypeStruct((E,), _f32),
                  jax.ShapeDtypeStruct((E,), _f32),
                  jax.ShapeDtypeStruct((E,), _f32)),
        mesh=mesh,
        scratch_types=scratch,
        compiler_params=pltpu.CompilerParams(needs_layout_passes=False),
    )


_views = _make_views()


BPW = B // NW


def _bgather_body(eu_t, ei_t, zu_t, zi_t, uids_h, iids_h, pos_h, neg_h,
                  o_eu, o_zu, o_ei, o_zi, o_ep, o_en, idx_v, row_v, sem):
    c = lax.axis_index("c")
    s = lax.axis_index("s")
    base = (s * NC + c) * BPW
    for tab, idxh, out in ((eu_t, uids_h, o_eu), (zu_t, uids_h, o_zu),
                           (ei_t, iids_h, o_ei), (zi_t, iids_h, o_zi),
                           (ei_t, pos_h, o_ep), (ei_t, neg_h, o_en)):
        pltpu.sync_copy(idxh.at[pl.ds(base, BPW)], idx_v)
        pltpu.async_copy(tab.at[idx_v], row_v, sem).wait()
        pltpu.sync_copy(row_v, out.at[pl.ds(base, BPW)])


def _make_bgather():
    mesh = plsc.VectorSubcoreMesh(core_axis_name="c", subcore_axis_name="s")
    bout = jax.ShapeDtypeStruct((B, D), _f32)
    return pl.kernel(
        _bgather_body,
        out_type=(bout,) * 6,
        mesh=mesh,
        scratch_types=[pltpu.VMEM((BPW,), _i32), pltpu.VMEM((BPW, D), _f32),
                       pltpu.SemaphoreType.DMA],
        compiler_params=pltpu.CompilerParams(needs_layout_passes=False),
    )


_bgather = _make_bgather()


EP = 2560 * 128
EB = 256
EGRID = EP // (EB * 128)
EVROWS = E // 128
NBLK = 128
NGRID = NPAD // NBLK


def _prep_body(eu0, ei0, wau, wai, hu, hi, sq, acc):
    pid = pl.program_id(0)

    @pl.when(pid == 0)
    def _():
        acc[0] = 0.0
        acc[1] = 0.0

    a = eu0[...]
    b = ei0[...]
    hu[...] = jnp.dot(a, wau[...], preferred_element_type=_f32)
    hi[...] = jnp.dot(b, wai[...], preferred_element_type=_f32)
    acc[0] += jnp.sum(a * a)
    acc[1] += jnp.sum(b * b)

    @pl.when(pid == NGRID - 1)
    def _():
        sq[0:1, :] = jnp.full((1, 128), acc[0], _f32)
        sq[1:2, :] = jnp.full((1, 128), acc[1], _f32)


def _prep(eu0p, ei0p, wau, wai):
    return pl.pallas_call(
        _prep_body,
        grid=(NGRID,),
        in_specs=[
            pl.BlockSpec((NBLK, D), lambda i: (i, 0)),
            pl.BlockSpec((NBLK, D), lambda i: (i, 0)),
            pl.BlockSpec((D, 2 * D), lambda i: (0, 0)),
            pl.BlockSpec((D, 2 * D), lambda i: (0, 0)),
        ],
        out_specs=[
            pl.BlockSpec((NBLK, 2 * D), lambda i: (i, 0)),
            pl.BlockSpec((NBLK, 2 * D), lambda i: (i, 0)),
            pl.BlockSpec((8, 128), lambda i: (0, 0)),
        ],
        out_shape=[jax.ShapeDtypeStruct((NPAD, 2 * D), _f32),
                   jax.ShapeDtypeStruct((NPAD, 2 * D), _f32),
                   jax.ShapeDtypeStruct((8, 128), _f32)],
        scratch_shapes=[pltpu.SMEM((2,), _f32)],
    )(eu0p, ei0p, wau, wai)


def _smallred_body(x, o):
    v = x[...]
    o[0:1, :] = jnp.full((1, 128), jnp.sum(v * v), _f32)


def _smallred(x):
    return pl.pallas_call(
        _smallred_body,
        out_shape=jax.ShapeDtypeStruct((8, 128), _f32),
    )(x)


def _combine2_body(p, o):
    o[...] = p[0] + p[1]


def _combine2(p):
    return pl.pallas_call(
        _combine2_body,
        grid=(NGRID,),
        in_specs=[pl.BlockSpec((2, NBLK, D), lambda i: (0, i, 0))],
        out_specs=pl.BlockSpec((NBLK, D), lambda i: (i, 0)),
        out_shape=jax.ShapeDtypeStruct((NPAD, D), _f32),
    )(p)


def _combine3_body(b0, a1, a2, o):
    o[...] = b0[...] + a1[...] + a2[...]


def _combine3(b0, a1, a2):
    nb = pl.BlockSpec((NBLK, D), lambda i: (i, 0))
    return pl.pallas_call(
        _combine3_body,
        grid=(NGRID,),
        in_specs=[nb, nb, nb],
        out_specs=nb,
        out_shape=jax.ShapeDtypeStruct((NPAD, D), _f32),
    )(b0, a1, a2)


def _sig(x):
    return 1.0 / (1.0 + jnp.exp(-x))


def _edgea_body(b2s, dg, dm, at, wv, fw, fb, n0, n1, n2, n3, ggcn, gsum,
                sums, acc):
    pid = pl.program_id(0)

    @pl.when(pid == 0)
    def _():
        for k in range(5):
            acc[k] = 0.0

    riota = lax.broadcasted_iota(_i32, (EB, 128), 0) + pid * EB
    valid = (riota < EVROWS).astype(_f32)
    fwv = fw[...]
    fbv = fb[...]
    gm = _sig(dm[...] + b2s[0])
    gw = _sig(wv[...])
    gg = _sig(dg[...])
    ga = _sig(at[...])
    outs = (n0, n1, n2, n3)
    for k, g in enumerate((gm, gw, gg, ga)):
        t = jnp.tanh(fwv * g + fbv)
        ek = jnp.exp(t)
        outs[k][...] = ek * g
        acc[k] += jnp.sum(ek * valid)
    ggcn[...] = gg
    gsum[...] = MLP_COF * gm + gw + gg + ga
    acc[4] += jnp.sum((fwv * fwv + fbv * fbv + wv[...] * wv[...]) * valid)

    @pl.when(pid == EGRID - 1)
    def _():
        for k in range(5):
            sums[k:k + 1, :] = jnp.full((1, 128), acc[k], _f32)


def _edgea(b2, dg, dm, at, wv, fw, fb):
    eb = pl.BlockSpec((EB, 128), lambda i: (i, 0))
    return pl.pallas_call(
        _edgea_body,
        grid=(EGRID,),
        in_specs=[pl.BlockSpec(memory_space=pltpu.SMEM)] + [eb] * 6,
        out_specs=[eb] * 6 + [pl.BlockSpec((8, 128), lambda i: (0, 0))],
        out_shape=[jax.ShapeDtypeStruct((EP // 128, 128), _f32)] * 6
        + [jax.ShapeDtypeStruct((8, 128), _f32)],
        scratch_shapes=[pltpu.SMEM((8,), _f32)],
    )(b2, dg, dm, at, wv, fw, fb)


def _edgeb_body(n0, n1, n2, n3, ggcn, gsum, adj, sums, aug, pr, acc):
    pid = pl.program_id(0)

    @pl.when(pid == 0)
    def _():
        acc[0] = 0.0

    agsoft = (n0[...] / sums[0:1, 0:1] + n1[...] / sums[1:2, 0:1]
              + n2[...] / sums[2:3, 0:1] + n3[...] / sums[3:4, 0:1])
    ag = (gsum[...] - (MLP_COF + 2.0) * agsoft) / (4.0 + MLP_COF)
    baw = ggcn[...] * ag
    aug[...] = baw * adj[...]
    riota = lax.broadcasted_iota(_i32, (EB, 128), 0) + pid * EB
    vmask = riota < EVROWS
    acc[0] += jnp.sum(jnp.where(vmask, -jnp.log(baw), 0.0))

    @pl.when(pid == EGRID - 1)
    def _():
        pr[0:1, :] = jnp.full((1, 128), acc[0], _f32)


def _edgeb(n0, n1, n2, n3, ggcn, gsum, adj, sums):
    eb = pl.BlockSpec((EB, 128), lambda i: (i, 0))
    s8 = pl.BlockSpec((8, 128), lambda i: (0, 0))
    return pl.pallas_call(
        _edgeb_body,
        grid=(EGRID,),
        in_specs=[eb] * 7 + [s8],
        out_specs=[eb, s8],
        out_shape=[jax.ShapeDtypeStruct((EP // 128, 128), _f32),
                   jax.ShapeDtypeStruct((8, 128), _f32)],
        scratch_shapes=[pltpu.SMEM((2,), _f32)],
    )(n0, n1, n2, n3, ggcn, gsum, adj, sums)


CB = 640
CGRID = NPAD // CB
NPADN = float(NPAD - N_U)


def _final_body(zu_b, eu_b, zi_b, ei_b, ep_b, en_b, eut, eit,
                sqp, sqs, sumsa, prb, out, accu, acci):
    pid = pl.program_id(0)

    @pl.when(pid == 0)
    def _():
        accu[...] = jnp.zeros_like(accu)
        acci[...] = jnp.zeros_like(acci)

    dn = (((1,), (1,)), ((), ()))
    lu = lax.dot_general(zu_b[...], eut[...], dn, preferred_element_type=_f32)
    li = lax.dot_general(zi_b[...], eit[...], dn, preferred_element_type=_f32)
    accu[...] += jnp.sum(jnp.exp(lu / TEMP), axis=1, keepdims=True)
    acci[...] += jnp.sum(jnp.exp(li / TEMP), axis=1, keepdims=True)

    @pl.when(pid == CGRID - 1)
    def _():
        su = accu[:, 0:1] - NPADN + 1e-08
        si = acci[:, 0:1] - NPADN + 1e-08
        neg_score = (jnp.sum(jnp.log(su)) + jnp.sum(jnp.log(si))) / B
        zu = zu_b[...]
        eu = eu_b[...]
        zi = zi_b[...]
        ei = ei_b[...]
        pos_u = jnp.clip(jnp.sum(zu * eu, axis=1) / TEMP, -5.0, 5.0)
        pos_i = jnp.clip(jnp.sum(zi * ei, axis=1) / TEMP, -5.0, 5.0)
        pos_score = (jnp.sum(pos_u) + jnp.sum(pos_i)) / B
        loss_cl = -pos_score + neg_score
        ps = jnp.sum(eu * ep_b[...], axis=1)
        ns = jnp.sum(eu * en_b[...], axis=1)
        loss_bpr = jnp.sum(-jnp.log(_sig(ps - ns))) / B
        loss_pr = LAMBDA_2 * prb[0, 0] / E
        reg = (sqp[0, 0] + sqp[1, 0] + sqs[0, 0] + sumsa[4, 0]) * LAMBDA_3
        loss = loss_bpr + LAMBDA_1 * loss_cl + loss_pr + reg
        out[0:1, :] = jnp.full((1, 128), loss, _f32)
        out[1:2, :] = jnp.full((1, 128), loss_bpr, _f32)
        out[2:3, :] = jnp.full((1, 128), LAMBDA_1 * loss_cl, _f32)
        out[3:4, :] = jnp.full((1, 128), loss_pr, _f32)


def _final(zu_b, eu_b, zi_b, ei_b, ep_b, en_b, eut, eit, sqp, sqs, sumsa, prb):
    bb = pl.BlockSpec((B, D), lambda i: (0, 0))
    nb = pl.BlockSpec((CB, D), lambda i: (i, 0))
    s8 = pl.BlockSpec((8, 128), lambda i: (0, 0))
    return pl.pallas_call(
        _final_body,
        grid=(CGRID,),
        in_specs=[bb] * 6 + [nb, nb] + [s8] * 4,
        out_specs=s8,
        out_shape=jax.ShapeDtypeStruct((8, 128), _f32),
        scratch_shapes=[pltpu.VMEM((B, 1), _f32), pltpu.VMEM((B, 1), _f32)],
    )(zu_b, eu_b, zi_b, ei_b, ep_b, en_b, eut, eit, sqp, sqs, sumsa, prb)


def _spmm_pair(ti, tu, src3, dst3, w2, scaled):
    f = _spmm_scaled if scaled else _spmm_plain
    pu, pi = f(ti, tu, src3, dst3, w2)
    return _combine2(pu), _combine2(pi)


def kernel(uids, iids, pos, neg, edge_index, adj_vals, E_u_0, E_i_0, fuse_w,
           fuse_b, wv_param, W1, b1, W2, b2, a_u, a_i):
    src = edge_index[0].astype(_i32)
    dst = edge_index[1].astype(_i32)
    src3 = src.reshape(NW, NCH // G, G, C)
    dst3 = dst.reshape(NW, NCH // G, G, C)


    Eu0p = jnp.zeros((NPAD, D), _f32).at[:N_U].set(E_u_0)
    Ei0p = jnp.zeros((NPAD, D), _f32).at[:N_I].set(E_i_0)

    w2col = W2[:, 0]
    wau = jnp.zeros((D, 2 * D), _f32).at[:, :D].set(W1[:D]).at[:, D].set(a_u)
    wai = jnp.zeros((D, 2 * D), _f32).at[:, :D].set(W1[D:]).at[:, D].set(a_i)
    HuP, HiP, sqp = _prep(Eu0p, Ei0p, wau, wai)
    smalls = jnp.zeros((264, 128), _f32)
    smalls = smalls.at[:256].set(W1)
    smalls = smalls.at[256].set(a_u).at[257].set(a_i)
    smalls = smalls.at[258].set(b1).at[259].set(w2col)
    smalls = smalls.at[260, 0].set(b2[0])
    sqs = _smallred(smalls)

    Eu1, Ei1 = _spmm_pair(Ei0p, Eu0p, src3, dst3, adj_vals, True)
    Eu2, Ei2 = _spmm_pair(Ei1, Eu1, src3, dst3, adj_vals, True)
    E_u = _combine3(Eu0p, Eu1, Eu2)
    E_i = _combine3(Ei0p, Ei1, Ei2)

    bw = jnp.stack([b1, w2col])
    dot_gcn, dot_mlp, attsum = _views(E_u, E_i, HuP[:, :D], HiP[:, :D], bw,
                                      HuP[:N_U, D], HiP[:N_I, D], src3, dst3)

    def pad2d(x):
        return jnp.zeros((EP,), _f32).at[:E].set(x).reshape(EP // 128, 128)

    n0, n1, n2, n3, ggcn2, gsum2, sumsa = _edgea(
        b2, pad2d(dot_gcn), pad2d(dot_mlp), pad2d(attsum), pad2d(wv_param),
        pad2d(fuse_w), pad2d(fuse_b))
    aug2, prb = _edgeb(n0, n1, n2, n3, ggcn2, gsum2, pad2d(adj_vals), sumsa)
    aug_vals = aug2.reshape(EP)[:E]

    Zu1, Zi1 = _spmm_pair(Ei0p, Eu0p, src3, dst3, aug_vals, True)
    Zu2, Zi2 = _spmm_pair(Zi1, Zu1, src3, dst3, aug_vals, True)
    Z_u = _combine3(Eu0p, Zu1, Zu2)
    Z_i = _combine3(Ei0p, Zi1, Zi2)

    ub = uids.astype(_i32)
    ib = iids.astype(_i32)
    pb_ = pos.astype(_i32)
    nb_ = neg.astype(_i32)
    o_eu, o_zu, o_ei, o_zi, o_ep, o_en = _bgather(E_u, E_i, Z_u, Z_i,
                                                  ub, ib, pb_, nb_)
    fin = _final(o_zu, o_eu, o_zi, o_ei, o_ep, o_en, E_u, E_i,
                 sqp, sqs, sumsa, prb)
    return fin[0, 0], fin[1, 0], fin[2, 0], fin[3, 0]


# speedup vs baseline: 1.1574x; 1.1574x over previous
import functools

import jax
import jax.numpy as jnp
from jax import lax
from jax.experimental import pallas as pl
from jax.experimental.pallas import tpu as pltpu
from jax.experimental.pallas import tpu_sc as plsc

N_U = 5000
N_I = 5000
NPAD = 5120
D = 128
E = 320000
TEMP = 0.2
LAMBDA_1 = 0.2
LAMBDA_2 = 1e-07
LAMBDA_3 = 1e-05
MLP_COF = 1.0
B = 1024

NC = 2
NS = 16
NW = NC * NS
EPW = E // NW
C = 40
NCH = EPW // C
RPT = NPAD // NS
G = 25

_f32 = jnp.float32
_i32 = jnp.int32


def _spmm_body(scaled, ti, tu, src3, dst3, w2, outu, outi,
               src_v, dst_v, w_v, rows_u, rows_i, zbuf, acc_u, acc_i,
               sem_u, sem_i, sem_u2, sem_i2):
    c = lax.axis_index("c")
    s = lax.axis_index("s")
    wid = s * NC + c

    zero = jnp.zeros((16,), _f32)
    for r in range(8):
        for k in range(8):
            zbuf[r, pl.ds(k * 16, 16)] = zero
    for b in range(RPT // 8):
        pltpu.sync_copy(zbuf, acc_u.at[pl.ds(s * RPT + b * 8, 8)])
        pltpu.sync_copy(zbuf, acc_i.at[pl.ds(s * RPT + b * 8, 8)])
    plsc.subcore_barrier()

    bufs_u = (rows_u.at[0], rows_u.at[1])
    bufs_i = (rows_i.at[0], rows_i.at[1])
    sems_u = (sem_u, sem_u2)
    sems_i = (sem_i, sem_i2)

    def fire(j, p):
        pltpu.async_copy(ti.at[dst_v.at[j]], bufs_u[p], sems_u[p])
        pltpu.async_copy(tu.at[src_v.at[j]], bufs_i[p], sems_i[p])

    def wait(j, p):
        pltpu.make_async_copy(ti.at[dst_v.at[j]], bufs_u[p], sems_u[p]).wait()
        pltpu.make_async_copy(tu.at[src_v.at[j]], bufs_i[p], sems_i[p]).wait()

    def process(j, p):
        bu, bi = bufs_u[p], bufs_i[p]
        if scaled:
            def edge(t, cc):
                for u in range(2):
                    e = t * 2 + u
                    wsp = plsc.load_gather(
                        w_v, [jnp.full((16,), j * C + e, _i32)])
                    for k in range(8):
                        sl = pl.ds(k * 16, 16)
                        bu[e, sl] = bu[e, sl] * wsp
                        bi[e, sl] = bi[e, sl] * wsp
                return cc
            lax.fori_loop(0, C // 2, edge, 0)
        pltpu.sync_copy(bu, acc_u.at[src_v.at[j]], add=True)
        pltpu.sync_copy(bi, acc_i.at[dst_v.at[j]], add=True)

    for g in range(NCH // G):
        pltpu.sync_copy(src3.at[wid, g], src_v)
        pltpu.sync_copy(dst3.at[wid, g], dst_v)
        if scaled:
            pltpu.sync_copy(w2.at[pl.ds(wid * EPW + g * G * C, G * C)], w_v)

        fire(0, 0)

        def pair(jj, carry):
            j0 = 2 * jj
            wait(j0, 0)
            fire(j0 + 1, 1)
            process(j0, 0)
            wait(j0 + 1, 1)
            fire(j0 + 2, 0)
            process(j0 + 1, 1)
            return carry

        lax.fori_loop(0, (G - 1) // 2, pair, 0)
        wait(G - 1, 0)
        process(G - 1, 0)
    plsc.subcore_barrier()

    pltpu.sync_copy(acc_u.at[pl.ds(s * RPT, RPT)], outu.at[c, pl.ds(s * RPT, RPT)])
    pltpu.sync_copy(acc_i.at[pl.ds(s * RPT, RPT)], outi.at[c, pl.ds(s * RPT, RPT)])


def _make_spmm(scaled):
    mesh = plsc.VectorSubcoreMesh(core_axis_name="c", subcore_axis_name="s")
    scratch = [
        pltpu.VMEM((G, C), _i32),
        pltpu.VMEM((G, C), _i32),
        pltpu.VMEM((G * C,), _f32),
        pltpu.VMEM((2, C, D), _f32),
        pltpu.VMEM((2, C, D), _f32),
        pltpu.VMEM((8, D), _f32),
        pltpu.VMEM_SHARED((NPAD, D), _f32),
        pltpu.VMEM_SHARED((NPAD, D), _f32),
        pltpu.SemaphoreType.DMA,
        pltpu.SemaphoreType.DMA,
        pltpu.SemaphoreType.DMA,
        pltpu.SemaphoreType.DMA,
    ]
    return pl.kernel(
        functools.partial(_spmm_body, scaled),
        out_type=(jax.ShapeDtypeStruct((NC, NPAD, D), _f32),
                  jax.ShapeDtypeStruct((NC, NPAD, D), _f32)),
        mesh=mesh,
        scratch_types=scratch,
        compiler_params=pltpu.CompilerParams(needs_layout_passes=False),
    )


_spmm_plain = _make_spmm(False)
_spmm_scaled = _make_spmm(True)


def _views_body(eu_t, ei_t, hu_t, hi_t, bw_t, pu_t, pi_t, src3, dst3,
                dgcn_o, dmlp_o, att_o,
                src_v, dst_v, eu_r, ei_r, hu_r, hi_r, bw_v, pu_v, pi_v,
                dg_v, dm_v, at_v, sem_a, sem_b, sem_c, sem_d):
    c = lax.axis_index("c")
    s = lax.axis_index("s")
    wid = s * NC + c

    pltpu.sync_copy(bw_t, bw_v)
    pltpu.sync_copy(pu_t, pu_v)
    pltpu.sync_copy(pi_t, pi_v)

    for g in range(NCH // G):
        pltpu.sync_copy(src3.at[wid, g], src_v)
        pltpu.sync_copy(dst3.at[wid, g], dst_v)

        def chunk(j, carry):
            pltpu.async_copy(eu_t.at[src_v.at[j]], eu_r, sem_a).wait()
            pltpu.async_copy(ei_t.at[dst_v.at[j]], ei_r, sem_b).wait()
            pltpu.async_copy(hu_t.at[src_v.at[j]], hu_r, sem_c).wait()
            pltpu.async_copy(hi_t.at[dst_v.at[j]], hi_r, sem_d).wait()

            def edge(e, cc):
                accg = jnp.zeros((16,), _f32)
                accm = jnp.zeros((16,), _f32)
                for k in range(8):
                    sl = pl.ds(k * 16, 16)
                    accg = accg + eu_r[e, sl] * ei_r[e, sl]
                    h = jnp.maximum(hu_r[e, sl] + hi_r[e, sl] + bw_v[0, sl], 0.0)
                    accm = accm + h * bw_v[1, sl]
                lane0 = lax.iota(_i32, 16) == 0
                eidx = jnp.full((16,), j * C + e, _i32)
                plsc.store_scatter(dg_v, [eidx],
                                   jnp.full((16,), jnp.sum(accg, axis=0), _f32),
                                   mask=lane0)
                plsc.store_scatter(dm_v, [eidx],
                                   jnp.full((16,), jnp.sum(accm, axis=0), _f32),
                                   mask=lane0)
                return cc
            lax.fori_loop(0, C, edge, 0)

            def att16(t, cc):
                si = src_v[j, pl.ds(t * 16, 16)]
                di = dst_v[j, pl.ds(t * 16, 16)]
                a = plsc.load_gather(pu_v, [si]) + plsc.load_gather(pi_v, [di])
                at_v[pl.ds(j * C + t * 16, 16)] = a
                return cc
            lax.fori_loop(0, C // 16, att16, 0)
            return carry

        lax.fori_loop(0, G, chunk, 0)
        base = wid * EPW + g * G * C
        pltpu.sync_copy(dg_v, dgcn_o.at[pl.ds(base, G * C)])
        pltpu.sync_copy(dm_v, dmlp_o.at[pl.ds(base, G * C)])
        pltpu.sync_copy(at_v, att_o.at[pl.ds(base, G * C)])


def _make_views():
    mesh = plsc.VectorSubcoreMesh(core_axis_name="c", subcore_axis_name="s")
    scratch = [
        pltpu.VMEM((G, C), _i32),
        pltpu.VMEM((G, C), _i32),
        pltpu.VMEM((C, D), _f32),
        pltpu.VMEM((C, D), _f32),
        pltpu.VMEM((C, D), _f32),
        pltpu.VMEM((C, D), _f32),
        pltpu.VMEM((2, D), _f32),
        pltpu.VMEM((N_U,), _f32),
        pltpu.VMEM((N_I,), _f32),
        pltpu.VMEM((G * C,), _f32),
        pltpu.VMEM((G * C,), _f32),
        pltpu.VMEM((G * C,), _f32),
        pltpu.SemaphoreType.DMA,
        pltpu.SemaphoreType.DMA,
        pltpu.SemaphoreType.DMA,
        pltpu.SemaphoreType.DMA,
    ]
    return pl.kernel(
        _views_body,
        out_type=(jax.ShapeDtypeStruct((E,), _f32),
                  jax.ShapeDtypeStruct((E,), _f32),
                  jax.ShapeDtypeStruct((E,), _f32)),
        mesh=mesh,
        scratch_types=scratch,
        compiler_params=pltpu.CompilerParams(needs_layout_passes=False),
    )


_views = _make_views()


BPW = B // NW


def _bgather_body(eu_t, ei_t, zu_t, zi_t, uids_h, iids_h, pos_h, neg_h,
                  o_eu, o_zu, o_ei, o_zi, o_ep, o_en, idx_v, row_v, sem):
    c = lax.axis_index("c")
    s = lax.axis_index("s")
    base = (s * NC + c) * BPW
    for tab, idxh, out in ((eu_t, uids_h, o_eu), (zu_t, uids_h, o_zu),
                           (ei_t, iids_h, o_ei), (zi_t, iids_h, o_zi),
                           (ei_t, pos_h, o_ep), (ei_t, neg_h, o_en)):
        pltpu.sync_copy(idxh.at[pl.ds(base, BPW)], idx_v)
        pltpu.async_copy(tab.at[idx_v], row_v, sem).wait()
        pltpu.sync_copy(row_v, out.at[pl.ds(base, BPW)])


def _make_bgather():
    mesh = plsc.VectorSubcoreMesh(core_axis_name="c", subcore_axis_name="s")
    bout = jax.ShapeDtypeStruct((B, D), _f32)
    return pl.kernel(
        _bgather_body,
        out_type=(bout,) * 6,
        mesh=mesh,
        scratch_types=[pltpu.VMEM((BPW,), _i32), pltpu.VMEM((BPW, D), _f32),
                       pltpu.SemaphoreType.DMA],
        compiler_params=pltpu.CompilerParams(needs_layout_passes=False),
    )


_bgather = _make_bgather()


EP = 2560 * 128
EB = 256
EGRID = EP // (EB * 128)
EVROWS = E // 128
NBLK = 128
NGRID = NPAD // NBLK


def _prep_body(eu0, ei0, wau, wai, hu, hi, sq, acc):
    pid = pl.program_id(0)

    @pl.when(pid == 0)
    def _():
        acc[0] = 0.0
        acc[1] = 0.0

    a = eu0[...]
    b = ei0[...]
    hu[...] = jnp.dot(a, wau[...], preferred_element_type=_f32)
    hi[...] = jnp.dot(b, wai[...], preferred_element_type=_f32)
    acc[0] += jnp.sum(a * a)
    acc[1] += jnp.sum(b * b)

    @pl.when(pid == NGRID - 1)
    def _():
        sq[0:1, :] = jnp.full((1, 128), acc[0], _f32)
        sq[1:2, :] = jnp.full((1, 128), acc[1], _f32)


def _prep(eu0p, ei0p, wau, wai):
    return pl.pallas_call(
        _prep_body,
        grid=(NGRID,),
        in_specs=[
            pl.BlockSpec((NBLK, D), lambda i: (i, 0)),
            pl.BlockSpec((NBLK, D), lambda i: (i, 0)),
            pl.BlockSpec((D, 2 * D), lambda i: (0, 0)),
            pl.BlockSpec((D, 2 * D), lambda i: (0, 0)),
        ],
        out_specs=[
            pl.BlockSpec((NBLK, 2 * D), lambda i: (i, 0)),
            pl.BlockSpec((NBLK, 2 * D), lambda i: (i, 0)),
            pl.BlockSpec((8, 128), lambda i: (0, 0)),
        ],
        out_shape=[jax.ShapeDtypeStruct((NPAD, 2 * D), _f32),
                   jax.ShapeDtypeStruct((NPAD, 2 * D), _f32),
                   jax.ShapeDtypeStruct((8, 128), _f32)],
        scratch_shapes=[pltpu.SMEM((2,), _f32)],
    )(eu0p, ei0p, wau, wai)


def _smallred_body(x, o):
    v = x[...]
    o[0:1, :] = jnp.full((1, 128), jnp.sum(v * v), _f32)


def _smallred(x):
    return pl.pallas_call(
        _smallred_body,
        out_shape=jax.ShapeDtypeStruct((8, 128), _f32),
    )(x)


def _combine2_body(p, o):
    o[...] = p[0] + p[1]


def _combine2(p):
    return pl.pallas_call(
        _combine2_body,
        grid=(NGRID,),
        in_specs=[pl.BlockSpec((2, NBLK, D), lambda i: (0, i, 0))],
        out_specs=pl.BlockSpec((NBLK, D), lambda i: (i, 0)),
        out_shape=jax.ShapeDtypeStruct((NPAD, D), _f32),
    )(p)


def _combine3_body(b0, a1, a2, o):
    o[...] = b0[...] + a1[...] + a2[...]


def _combine3(b0, a1, a2):
    nb = pl.BlockSpec((NBLK, D), lambda i: (i, 0))
    return pl.pallas_call(
        _combine3_body,
        grid=(NGRID,),
        in_specs=[nb, nb, nb],
        out_specs=nb,
        out_shape=jax.ShapeDtypeStruct((NPAD, D), _f32),
    )(b0, a1, a2)


def _sig(x):
    return 1.0 / (1.0 + jnp.exp(-x))


def _edgea_body(b2s, dg, dm, at, wv, fw, fb, n0, n1, n2, n3, ggcn, gsum,
                sums, acc):
    pid = pl.program_id(0)

    @pl.when(pid == 0)
    def _():
        for k in range(5):
            acc[k] = 0.0

    riota = lax.broadcasted_iota(_i32, (EB, 128), 0) + pid * EB
    valid = (riota < EVROWS).astype(_f32)
    fwv = fw[...]
    fbv = fb[...]
    gm = _sig(dm[...] + b2s[0])
    gw = _sig(wv[...])
    gg = _sig(dg[...])
    ga = _sig(at[...])
    outs = (n0, n1, n2, n3)
    for k, g in enumerate((gm, gw, gg, ga)):
        t = jnp.tanh(fwv * g + fbv)
        ek = jnp.exp(t)
        outs[k][...] = ek * g
        acc[k] += jnp.sum(ek * valid)
    ggcn[...] = gg
    gsum[...] = MLP_COF * gm + gw + gg + ga
    acc[4] += jnp.sum((fwv * fwv + fbv * fbv + wv[...] * wv[...]) * valid)

    @pl.when(pid == EGRID - 1)
    def _():
        for k in range(5):
            sums[k:k + 1, :] = jnp.full((1, 128), acc[k], _f32)


def _edgea(b2, dg, dm, at, wv, fw, fb):
    eb = pl.BlockSpec((EB, 128), lambda i: (i, 0))
    return pl.pallas_call(
        _edgea_body,
        grid=(EGRID,),
        in_specs=[pl.BlockSpec(memory_space=pltpu.SMEM)] + [eb] * 6,
        out_specs=[eb] * 6 + [pl.BlockSpec((8, 128), lambda i: (0, 0))],
        out_shape=[jax.ShapeDtypeStruct((EP // 128, 128), _f32)] * 6
        + [jax.ShapeDtypeStruct((8, 128), _f32)],
        scratch_shapes=[pltpu.SMEM((8,), _f32)],
    )(b2, dg, dm, at, wv, fw, fb)


def _edgeb_body(n0, n1, n2, n3, ggcn, gsum, adj, sums, aug, pr, acc):
    pid = pl.program_id(0)

    @pl.when(pid == 0)
    def _():
        acc[0] = 0.0

    agsoft = (n0[...] / sums[0:1, 0:1] + n1[...] / sums[1:2, 0:1]
              + n2[...] / sums[2:3, 0:1] + n3[...] / sums[3:4, 0:1])
    ag = (gsum[...] - (MLP_COF + 2.0) * agsoft) / (4.0 + MLP_COF)
    baw = ggcn[...] * ag
    aug[...] = baw * adj[...]
    riota = lax.broadcasted_iota(_i32, (EB, 128), 0) + pid * EB
    vmask = riota < EVROWS
    acc[0] += jnp.sum(jnp.where(vmask, -jnp.log(baw), 0.0))

    @pl.when(pid == EGRID - 1)
    def _():
        pr[0:1, :] = jnp.full((1, 128), acc[0], _f32)


def _edgeb(n0, n1, n2, n3, ggcn, gsum, adj, sums):
    eb = pl.BlockSpec((EB, 128), lambda i: (i, 0))
    s8 = pl.BlockSpec((8, 128), lambda i: (0, 0))
    return pl.pallas_call(
        _edgeb_body,
        grid=(EGRID,),
        in_specs=[eb] * 7 + [s8],
        out_specs=[eb, s8],
        out_shape=[jax.ShapeDtypeStruct((EP // 128, 128), _f32),
                   jax.ShapeDtypeStruct((8, 128), _f32)],
        scratch_shapes=[pltpu.SMEM((2,), _f32)],
    )(n0, n1, n2, n3, ggcn, gsum, adj, sums)


CB = 640
CGRID = NPAD // CB
NPADN = float(NPAD - N_U)


def _final_body(zu_b, eu_b, zi_b, ei_b, ep_b, en_b, eut, eit,
                sqp, sqs, sumsa, prb, out, accu, acci):
    pid = pl.program_id(0)

    @pl.when(pid == 0)
    def _():
        accu[...] = jnp.zeros_like(accu)
        acci[...] = jnp.zeros_like(acci)

    dn = (((1,), (1,)), ((), ()))
    lu = lax.dot_general(zu_b[...], eut[...], dn, preferred_element_type=_f32)
    li = lax.dot_general(zi_b[...], eit[...], dn, preferred_element_type=_f32)
    accu[...] += jnp.sum(jnp.exp(lu / TEMP), axis=1, keepdims=True)
    acci[...] += jnp.sum(jnp.exp(li / TEMP), axis=1, keepdims=True)

    @pl.when(pid == CGRID - 1)
    def _():
        su = accu[:, 0:1] - NPADN + 1e-08
        si = acci[:, 0:1] - NPADN + 1e-08
        neg_score = (jnp.sum(jnp.log(su)) + jnp.sum(jnp.log(si))) / B
        zu = zu_b[...]
        eu = eu_b[...]
        zi = zi_b[...]
        ei = ei_b[...]
        pos_u = jnp.clip(jnp.sum(zu * eu, axis=1) / TEMP, -5.0, 5.0)
        pos_i = jnp.clip(jnp.sum(zi * ei, axis=1) / TEMP, -5.0, 5.0)
        pos_score = (jnp.sum(pos_u) + jnp.sum(pos_i)) / B
        loss_cl = -pos_score + neg_score
        ps = jnp.sum(eu * ep_b[...], axis=1)
        ns = jnp.sum(eu * en_b[...], axis=1)
        loss_bpr = jnp.sum(-jnp.log(_sig(ps - ns))) / B
        loss_pr = LAMBDA_2 * prb[0, 0] / E
        reg = (sqp[0, 0] + sqp[1, 0] + sqs[0, 0] + sumsa[4, 0]) * LAMBDA_3
        loss = loss_bpr + LAMBDA_1 * loss_cl + loss_pr + reg
        out[0:1, :] = jnp.full((1, 128), loss, _f32)
        out[1:2, :] = jnp.full((1, 128), loss_bpr, _f32)
        out[2:3, :] = jnp.full((1, 128), LAMBDA_1 * loss_cl, _f32)
        out[3:4, :] = jnp.full((1, 128), loss_pr, _f32)


def _final(zu_b, eu_b, zi_b, ei_b, ep_b, en_b, eut, eit, sqp, sqs, sumsa, prb):
    bb = pl.BlockSpec((B, D), lambda i: (0, 0))
    nb = pl.BlockSpec((CB, D), lambda i: (i, 0))
    s8 = pl.BlockSpec((8, 128), lambda i: (0, 0))
    return pl.pallas_call(
        _final_body,
        grid=(CGRID,),
        in_specs=[bb] * 6 + [nb, nb] + [s8] * 4,
        out_specs=s8,
        out_shape=jax.ShapeDtypeStruct((8, 128), _f32),
        scratch_shapes=[pltpu.VMEM((B, 1), _f32), pltpu.VMEM((B, 1), _f32)],
    )(zu_b, eu_b, zi_b, ei_b, ep_b, en_b, eut, eit, sqp, sqs, sumsa, prb)


def _spmm_pair(ti, tu, src3, dst3, w2, scaled):
    f = _spmm_scaled if scaled else _spmm_plain
    pu, pi = f(ti, tu, src3, dst3, w2)
    return _combine2(pu), _combine2(pi)


def kernel(uids, iids, pos, neg, edge_index, adj_vals, E_u_0, E_i_0, fuse_w,
           fuse_b, wv_param, W1, b1, W2, b2, a_u, a_i):
    src = edge_index[0].astype(_i32)
    dst = edge_index[1].astype(_i32)
    src3 = src.reshape(NW, NCH // G, G, C)
    dst3 = dst.reshape(NW, NCH // G, G, C)


    Eu0p = jnp.zeros((NPAD, D), _f32).at[:N_U].set(E_u_0)
    Ei0p = jnp.zeros((NPAD, D), _f32).at[:N_I].set(E_i_0)

    w2col = W2[:, 0]
    wau = jnp.zeros((D, 2 * D), _f32).at[:, :D].set(W1[:D]).at[:, D].set(a_u)
    wai = jnp.zeros((D, 2 * D), _f32).at[:, :D].set(W1[D:]).at[:, D].set(a_i)
    HuP, HiP, sqp = _prep(Eu0p, Ei0p, wau, wai)
    smalls = jnp.zeros((264, 128), _f32)
    smalls = smalls.at[:256].set(W1)
    smalls = smalls.at[256].set(a_u).at[257].set(a_i)
    smalls = smalls.at[258].set(b1).at[259].set(w2col)
    smalls = smalls.at[260, 0].set(b2[0])
    sqs = _smallred(smalls)

    Eu1, Ei1 = _spmm_pair(Ei0p, Eu0p, src3, dst3, adj_vals, True)
    Eu2, Ei2 = _spmm_pair(Ei1, Eu1, src3, dst3, adj_vals, True)
    E_u = _combine3(Eu0p, Eu1, Eu2)
    E_i = _combine3(Ei0p, Ei1, Ei2)

    bw = jnp.stack([b1, w2col])
    dot_gcn, dot_mlp, attsum = _views(E_u, E_i, HuP[:, :D], HiP[:, :D], bw,
                                      HuP[:N_U, D], HiP[:N_I, D], src3, dst3)

    def pad2d(x):
        return jnp.zeros((EP,), _f32).at[:E].set(x).reshape(EP // 128, 128)

    n0, n1, n2, n3, ggcn2, gsum2, sumsa = _edgea(
        b2, pad2d(dot_gcn), pad2d(dot_mlp), pad2d(attsum), pad2d(wv_param),
        pad2d(fuse_w), pad2d(fuse_b))
    aug2, prb = _edgeb(n0, n1, n2, n3, ggcn2, gsum2, pad2d(adj_vals), sumsa)
    aug_vals = aug2.reshape(EP)[:E]

    Zu1, Zi1 = _spmm_pair(Ei0p, Eu0p, src3, dst3, aug_vals, True)
    Zu2, Zi2 = _spmm_pair(Zi1, Zu1, src3, dst3, aug_vals, True)
    Z_u = _combine3(Eu0p, Zu1, Zu2)
    Z_i = _combine3(Ei0p, Zi1, Zi2)

    ub = uids.astype(_i32)
    ib = iids.astype(_i32)
    pb_ = pos.astype(_i32)
    nb_ = neg.astype(_i32)
    o_eu, o_zu, o_ei, o_zi, o_ep, o_en = _bgather(E_u, E_i, Z_u, Z_i,
                                                  ub, ib, pb_, nb_)
    fin = _final(o_zu, o_eu, o_zi, o_ei, o_ep, o_en, E_u, E_i,
                 sqp, sqs, sumsa, prb)
    return fin[0, 0], fin[1, 0], fin[2, 0], fin[3, 0]


# speedup vs baseline: 1.5983x; 1.3810x over previous
import functools

import jax
import jax.numpy as jnp
from jax import lax
from jax.experimental import pallas as pl
from jax.experimental.pallas import tpu as pltpu
from jax.experimental.pallas import tpu_sc as plsc

N_U = 5000
N_I = 5000
NPAD = 5120
D = 128
E = 320000
TEMP = 0.2
LAMBDA_1 = 0.2
LAMBDA_2 = 1e-07
LAMBDA_3 = 1e-05
MLP_COF = 1.0
B = 1024

NC = 2
NS = 16
NW = NC * NS
EPW = E // NW
C = 40
NCH = EPW // C
RPT = NPAD // NS
G = 25

_f32 = jnp.float32
_i32 = jnp.int32


def _spmm_body(scaled, ti, tu, src3, dst3, w2, outu, outi,
               src_v, dst_v, w_v, rows_u, rows_i, zbuf, acc_u, acc_i,
               sem_u, sem_i, sem_u2, sem_i2):
    c = lax.axis_index("c")
    s = lax.axis_index("s")
    wid = s * NC + c

    zero = jnp.zeros((16,), _f32)
    for r in range(8):
        for k in range(8):
            zbuf[r, pl.ds(k * 16, 16)] = zero
    for b in range(RPT // 8):
        pltpu.sync_copy(zbuf, acc_u.at[pl.ds(s * RPT + b * 8, 8)])
        pltpu.sync_copy(zbuf, acc_i.at[pl.ds(s * RPT + b * 8, 8)])
    plsc.subcore_barrier()

    bufs_u = (rows_u.at[0], rows_u.at[1])
    bufs_i = (rows_i.at[0], rows_i.at[1])
    sems_u = (sem_u, sem_u2)
    sems_i = (sem_i, sem_i2)

    def fire(j, p):
        pltpu.async_copy(ti.at[dst_v.at[j]], bufs_u[p], sems_u[p])
        pltpu.async_copy(tu.at[src_v.at[j]], bufs_i[p], sems_i[p])

    def wait(j, p):
        pltpu.make_async_copy(ti.at[dst_v.at[j]], bufs_u[p], sems_u[p]).wait()
        pltpu.make_async_copy(tu.at[src_v.at[j]], bufs_i[p], sems_i[p]).wait()

    def process(j, p):
        bu, bi = bufs_u[p], bufs_i[p]
        if scaled:
            def edge(t, cc):
                for u in range(2):
                    e = t * 2 + u
                    wsp = plsc.load_gather(
                        w_v, [jnp.full((16,), j * C + e, _i32)])
                    for k in range(8):
                        sl = pl.ds(k * 16, 16)
                        bu[e, sl] = bu[e, sl] * wsp
                        bi[e, sl] = bi[e, sl] * wsp
                return cc
            lax.fori_loop(0, C // 2, edge, 0)
        pltpu.sync_copy(bu, acc_u.at[src_v.at[j]], add=True)
        pltpu.sync_copy(bi, acc_i.at[dst_v.at[j]], add=True)

    for g in range(NCH // G):
        pltpu.sync_copy(src3.at[wid, g], src_v)
        pltpu.sync_copy(dst3.at[wid, g], dst_v)
        if scaled:
            pltpu.sync_copy(w2.at[pl.ds(wid * EPW + g * G * C, G * C)], w_v)

        fire(0, 0)

        def pair(jj, carry):
            j0 = 2 * jj
            wait(j0, 0)
            fire(j0 + 1, 1)
            process(j0, 0)
            wait(j0 + 1, 1)
            fire(j0 + 2, 0)
            process(j0 + 1, 1)
            return carry

        lax.fori_loop(0, (G - 1) // 2, pair, 0)
        wait(G - 1, 0)
        process(G - 1, 0)
    plsc.subcore_barrier()

    pltpu.sync_copy(acc_u.at[pl.ds(s * RPT, RPT)], outu.at[c, pl.ds(s * RPT, RPT)])
    pltpu.sync_copy(acc_i.at[pl.ds(s * RPT, RPT)], outi.at[c, pl.ds(s * RPT, RPT)])


def _make_spmm(scaled):
    mesh = plsc.VectorSubcoreMesh(core_axis_name="c", subcore_axis_name="s")
    scratch = [
        pltpu.VMEM((G, C), _i32),
        pltpu.VMEM((G, C), _i32),
        pltpu.VMEM((G * C,), _f32),
        pltpu.VMEM((2, C, D), _f32),
        pltpu.VMEM((2, C, D), _f32),
        pltpu.VMEM((8, D), _f32),
        pltpu.VMEM_SHARED((NPAD, D), _f32),
        pltpu.VMEM_SHARED((NPAD, D), _f32),
        pltpu.SemaphoreType.DMA,
        pltpu.SemaphoreType.DMA,
        pltpu.SemaphoreType.DMA,
        pltpu.SemaphoreType.DMA,
    ]
    return pl.kernel(
        functools.partial(_spmm_body, scaled),
        out_type=(jax.ShapeDtypeStruct((NC, NPAD, D), _f32),
                  jax.ShapeDtypeStruct((NC, NPAD, D), _f32)),
        mesh=mesh,
        scratch_types=scratch,
        compiler_params=pltpu.CompilerParams(needs_layout_passes=False),
    )


_spmm_plain = _make_spmm(False)
_spmm_scaled = _make_spmm(True)


CV = 80
GV = 25
NGRV = EPW // (GV * CV)


def _views_body(eu_t, ei_t, hu_t, hi_t, bw_t, pu_t, pi_t, src3, dst3,
                dgcn_o, dmlp_o, att_o,
                src_v, dst_v, eu_r, ei_r, hu_r, hi_r, bw_v, pu_v, pi_v,
                dg_v, dm_v, at_v, *sems):
    c = lax.axis_index("c")
    s = lax.axis_index("s")
    wid = s * NC + c

    pltpu.sync_copy(bw_t, bw_v)
    pltpu.sync_copy(pu_t, pu_v)
    pltpu.sync_copy(pi_t, pi_v)

    def fire(j, p):
        pltpu.async_copy(eu_t.at[src_v.at[j]], eu_r.at[p], sems[p * 4])
        pltpu.async_copy(ei_t.at[dst_v.at[j]], ei_r.at[p], sems[p * 4 + 1])
        pltpu.async_copy(hu_t.at[src_v.at[j]], hu_r.at[p], sems[p * 4 + 2])
        pltpu.async_copy(hi_t.at[dst_v.at[j]], hi_r.at[p], sems[p * 4 + 3])

    def wait(j, p):
        pltpu.make_async_copy(eu_t.at[src_v.at[j]], eu_r.at[p], sems[p * 4]).wait()
        pltpu.make_async_copy(ei_t.at[dst_v.at[j]], ei_r.at[p], sems[p * 4 + 1]).wait()
        pltpu.make_async_copy(hu_t.at[src_v.at[j]], hu_r.at[p], sems[p * 4 + 2]).wait()
        pltpu.make_async_copy(hi_t.at[dst_v.at[j]], hi_r.at[p], sems[p * 4 + 3]).wait()

    def process(j, p):
        eu_b, ei_b, hu_b, hi_b = eu_r.at[p], ei_r.at[p], hu_r.at[p], hi_r.at[p]

        def edge(t, cc):
            lane0 = lax.iota(_i32, 16) == 0
            for u in range(2):
                e = t * 2 + u
                accg = jnp.zeros((16,), _f32)
                accm = jnp.zeros((16,), _f32)
                for k in range(8):
                    sl = pl.ds(k * 16, 16)
                    accg = accg + eu_b[e, sl] * ei_b[e, sl]
                    h = jnp.maximum(hu_b[e, sl] + hi_b[e, sl] + bw_v[0, sl], 0.0)
                    accm = accm + h * bw_v[1, sl]
                eidx = jnp.full((16,), j * CV + e, _i32)
                plsc.store_scatter(dg_v, [eidx],
                                   jnp.full((16,), jnp.sum(accg, axis=0), _f32),
                                   mask=lane0)
                plsc.store_scatter(dm_v, [eidx],
                                   jnp.full((16,), jnp.sum(accm, axis=0), _f32),
                                   mask=lane0)
            return cc
        lax.fori_loop(0, CV // 2, edge, 0)

        def att16(t, cc):
            si = src_v[j, pl.ds(t * 16, 16)]
            di = dst_v[j, pl.ds(t * 16, 16)]
            a = plsc.load_gather(pu_v, [si]) + plsc.load_gather(pi_v, [di])
            at_v[pl.ds(j * CV + t * 16, 16)] = a
            return cc
        lax.fori_loop(0, CV // 16, att16, 0)

    for g in range(NGRV):
        pltpu.sync_copy(src3.at[wid, g], src_v)
        pltpu.sync_copy(dst3.at[wid, g], dst_v)
        fire(0, 0)

        def pair(jj, carry):
            j0 = 2 * jj
            wait(j0, 0)
            fire(j0 + 1, 1)
            process(j0, 0)
            wait(j0 + 1, 1)
            fire(j0 + 2, 0)
            process(j0 + 1, 1)
            return carry

        lax.fori_loop(0, (GV - 1) // 2, pair, 0)
        wait(GV - 1, 0)
        process(GV - 1, 0)
        base = wid * EPW + g * GV * CV
        pltpu.sync_copy(dg_v, dgcn_o.at[pl.ds(base, GV * CV)])
        pltpu.sync_copy(dm_v, dmlp_o.at[pl.ds(base, GV * CV)])
        pltpu.sync_copy(at_v, att_o.at[pl.ds(base, GV * CV)])


def _make_views():
    mesh = plsc.VectorSubcoreMesh(core_axis_name="c", subcore_axis_name="s")
    scratch = [
        pltpu.VMEM((GV, CV), _i32),
        pltpu.VMEM((GV, CV), _i32),
        pltpu.VMEM((2, CV, D), _f32),
        pltpu.VMEM((2, CV, D), _f32),
        pltpu.VMEM((2, CV, D), _f32),
        pltpu.VMEM((2, CV, D), _f32),
        pltpu.VMEM((2, D), _f32),
        pltpu.VMEM((N_U,), _f32),
        pltpu.VMEM((N_I,), _f32),
        pltpu.VMEM((GV * CV,), _f32),
        pltpu.VMEM((GV * CV,), _f32),
        pltpu.VMEM((GV * CV,), _f32),
    ] + [pltpu.SemaphoreType.DMA] * 8
    return pl.kernel(
        _views_body,
        out_type=(jax.ShapeDtypeStruct((E,), _f32),
                  jax.ShapeDtypeStruct((E,), _f32),
                  jax.ShapeDtypeStruct((E,), _f32)),
        mesh=mesh,
        scratch_types=scratch,
        compiler_params=pltpu.CompilerParams(needs_layout_passes=False),
    )


_views = _make_views()


BPW = B // NW


def _bgather_body(eu_t, ei_t, zu_t, zi_t, uids_h, iids_h, pos_h, neg_h,
                  o_eu, o_zu, o_ei, o_zi, o_ep, o_en, idx_v, row_v, sem):
    c = lax.axis_index("c")
    s = lax.axis_index("s")
    base = (s * NC + c) * BPW
    for tab, idxh, out in ((eu_t, uids_h, o_eu), (zu_t, uids_h, o_zu),
                           (ei_t, iids_h, o_ei), (zi_t, iids_h, o_zi),
                           (ei_t, pos_h, o_ep), (ei_t, neg_h, o_en)):
        pltpu.sync_copy(idxh.at[pl.ds(base, BPW)], idx_v)
        pltpu.async_copy(tab.at[idx_v], row_v, sem).wait()
        pltpu.sync_copy(row_v, out.at[pl.ds(base, BPW)])


def _make_bgather():
    mesh = plsc.VectorSubcoreMesh(core_axis_name="c", subcore_axis_name="s")
    bout = jax.ShapeDtypeStruct((B, D), _f32)
    return pl.kernel(
        _bgather_body,
        out_type=(bout,) * 6,
        mesh=mesh,
        scratch_types=[pltpu.VMEM((BPW,), _i32), pltpu.VMEM((BPW, D), _f32),
                       pltpu.SemaphoreType.DMA],
        compiler_params=pltpu.CompilerParams(needs_layout_passes=False),
    )


_bgather = _make_bgather()


EP = 2560 * 128
EB = 256
EGRID = EP // (EB * 128)
EVROWS = E // 128
NBLK = 128
NGRID = NPAD // NBLK


def _prep_body(eu0, ei0, wau, wai, hu, hi, sq, acc):
    pid = pl.program_id(0)

    @pl.when(pid == 0)
    def _():
        acc[0] = 0.0
        acc[1] = 0.0

    a = eu0[...]
    b = ei0[...]
    hu[...] = jnp.dot(a, wau[...], preferred_element_type=_f32)
    hi[...] = jnp.dot(b, wai[...], preferred_element_type=_f32)
    acc[0] += jnp.sum(a * a)
    acc[1] += jnp.sum(b * b)

    @pl.when(pid == NGRID - 1)
    def _():
        sq[0:1, :] = jnp.full((1, 128), acc[0], _f32)
        sq[1:2, :] = jnp.full((1, 128), acc[1], _f32)


def _prep(eu0p, ei0p, wau, wai):
    return pl.pallas_call(
        _prep_body,
        grid=(NGRID,),
        in_specs=[
            pl.BlockSpec((NBLK, D), lambda i: (i, 0)),
            pl.BlockSpec((NBLK, D), lambda i: (i, 0)),
            pl.BlockSpec((D, 2 * D), lambda i: (0, 0)),
            pl.BlockSpec((D, 2 * D), lambda i: (0, 0)),
        ],
        out_specs=[
            pl.BlockSpec((NBLK, 2 * D), lambda i: (i, 0)),
            pl.BlockSpec((NBLK, 2 * D), lambda i: (i, 0)),
            pl.BlockSpec((8, 128), lambda i: (0, 0)),
        ],
        out_shape=[jax.ShapeDtypeStruct((NPAD, 2 * D), _f32),
                   jax.ShapeDtypeStruct((NPAD, 2 * D), _f32),
                   jax.ShapeDtypeStruct((8, 128), _f32)],
        scratch_shapes=[pltpu.SMEM((2,), _f32)],
    )(eu0p, ei0p, wau, wai)


def _smallred_body(x, o):
    v = x[...]
    o[0:1, :] = jnp.full((1, 128), jnp.sum(v * v), _f32)


def _smallred(x):
    return pl.pallas_call(
        _smallred_body,
        out_shape=jax.ShapeDtypeStruct((8, 128), _f32),
    )(x)


def _combine2_body(p, o):
    o[...] = p[0] + p[1]


def _combine2(p):
    return pl.pallas_call(
        _combine2_body,
        grid=(NGRID,),
        in_specs=[pl.BlockSpec((2, NBLK, D), lambda i: (0, i, 0))],
        out_specs=pl.BlockSpec((NBLK, D), lambda i: (i, 0)),
        out_shape=jax.ShapeDtypeStruct((NPAD, D), _f32),
    )(p)


def _combine3_body(b0, a1, a2, o):
    o[...] = b0[...] + a1[...] + a2[...]


def _combine3(b0, a1, a2):
    nb = pl.BlockSpec((NBLK, D), lambda i: (i, 0))
    return pl.pallas_call(
        _combine3_body,
        grid=(NGRID,),
        in_specs=[nb, nb, nb],
        out_specs=nb,
        out_shape=jax.ShapeDtypeStruct((NPAD, D), _f32),
    )(b0, a1, a2)


def _sig(x):
    return 1.0 / (1.0 + jnp.exp(-x))


def _edgea_body(b2s, dg, dm, at, wv, fw, fb, n0, n1, n2, n3, ggcn, gsum,
                sums, acc):
    pid = pl.program_id(0)

    @pl.when(pid == 0)
    def _():
        for k in range(5):
            acc[k] = 0.0

    riota = lax.broadcasted_iota(_i32, (EB, 128), 0) + pid * EB
    valid = (riota < EVROWS).astype(_f32)
    fwv = fw[...]
    fbv = fb[...]
    gm = _sig(dm[...] + b2s[0])
    gw = _sig(wv[...])
    gg = _sig(dg[...])
    ga = _sig(at[...])
    outs = (n0, n1, n2, n3)
    for k, g in enumerate((gm, gw, gg, ga)):
        t = jnp.tanh(fwv * g + fbv)
        ek = jnp.exp(t)
        outs[k][...] = ek * g
        acc[k] += jnp.sum(ek * valid)
    ggcn[...] = gg
    gsum[...] = MLP_COF * gm + gw + gg + ga
    acc[4] += jnp.sum((fwv * fwv + fbv * fbv + wv[...] * wv[...]) * valid)

    @pl.when(pid == EGRID - 1)
    def _():
        for k in range(5):
            sums[k:k + 1, :] = jnp.full((1, 128), acc[k], _f32)


def _edgea(b2, dg, dm, at, wv, fw, fb):
    eb = pl.BlockSpec((EB, 128), lambda i: (i, 0))
    return pl.pallas_call(
        _edgea_body,
        grid=(EGRID,),
        in_specs=[pl.BlockSpec(memory_space=pltpu.SMEM)] + [eb] * 6,
        out_specs=[eb] * 6 + [pl.BlockSpec((8, 128), lambda i: (0, 0))],
        out_shape=[jax.ShapeDtypeStruct((EP // 128, 128), _f32)] * 6
        + [jax.ShapeDtypeStruct((8, 128), _f32)],
        scratch_shapes=[pltpu.SMEM((8,), _f32)],
    )(b2, dg, dm, at, wv, fw, fb)


def _edgeb_body(n0, n1, n2, n3, ggcn, gsum, adj, sums, aug, pr, acc):
    pid = pl.program_id(0)

    @pl.when(pid == 0)
    def _():
        acc[0] = 0.0

    agsoft = (n0[...] / sums[0:1, 0:1] + n1[...] / sums[1:2, 0:1]
              + n2[...] / sums[2:3, 0:1] + n3[...] / sums[3:4, 0:1])
    ag = (gsum[...] - (MLP_COF + 2.0) * agsoft) / (4.0 + MLP_COF)
    baw = ggcn[...] * ag
    aug[...] = baw * adj[...]
    riota = lax.broadcasted_iota(_i32, (EB, 128), 0) + pid * EB
    vmask = riota < EVROWS
    acc[0] += jnp.sum(jnp.where(vmask, -jnp.log(baw), 0.0))

    @pl.when(pid == EGRID - 1)
    def _():
        pr[0:1, :] = jnp.full((1, 128), acc[0], _f32)


def _edgeb(n0, n1, n2, n3, ggcn, gsum, adj, sums):
    eb = pl.BlockSpec((EB, 128), lambda i: (i, 0))
    s8 = pl.BlockSpec((8, 128), lambda i: (0, 0))
    return pl.pallas_call(
        _edgeb_body,
        grid=(EGRID,),
        in_specs=[eb] * 7 + [s8],
        out_specs=[eb, s8],
        out_shape=[jax.ShapeDtypeStruct((EP // 128, 128), _f32),
                   jax.ShapeDtypeStruct((8, 128), _f32)],
        scratch_shapes=[pltpu.SMEM((2,), _f32)],
    )(n0, n1, n2, n3, ggcn, gsum, adj, sums)


CB = 640
CGRID = NPAD // CB
NPADN = float(NPAD - N_U)


def _final_body(zu_b, eu_b, zi_b, ei_b, ep_b, en_b, eut, eit,
                sqp, sqs, sumsa, prb, out, accu, acci):
    pid = pl.program_id(0)

    @pl.when(pid == 0)
    def _():
        accu[...] = jnp.zeros_like(accu)
        acci[...] = jnp.zeros_like(acci)

    dn = (((1,), (1,)), ((), ()))
    lu = lax.dot_general(zu_b[...], eut[...], dn, preferred_element_type=_f32)
    li = lax.dot_general(zi_b[...], eit[...], dn, preferred_element_type=_f32)
    accu[...] += jnp.sum(jnp.exp(lu / TEMP), axis=1, keepdims=True)
    acci[...] += jnp.sum(jnp.exp(li / TEMP), axis=1, keepdims=True)

    @pl.when(pid == CGRID - 1)
    def _():
        su = accu[:, 0:1] - NPADN + 1e-08
        si = acci[:, 0:1] - NPADN + 1e-08
        neg_score = (jnp.sum(jnp.log(su)) + jnp.sum(jnp.log(si))) / B
        zu = zu_b[...]
        eu = eu_b[...]
        zi = zi_b[...]
        ei = ei_b[...]
        pos_u = jnp.clip(jnp.sum(zu * eu, axis=1) / TEMP, -5.0, 5.0)
        pos_i = jnp.clip(jnp.sum(zi * ei, axis=1) / TEMP, -5.0, 5.0)
        pos_score = (jnp.sum(pos_u) + jnp.sum(pos_i)) / B
        loss_cl = -pos_score + neg_score
        ps = jnp.sum(eu * ep_b[...], axis=1)
        ns = jnp.sum(eu * en_b[...], axis=1)
        loss_bpr = jnp.sum(-jnp.log(_sig(ps - ns))) / B
        loss_pr = LAMBDA_2 * prb[0, 0] / E
        reg = (sqp[0, 0] + sqp[1, 0] + sqs[0, 0] + sumsa[4, 0]) * LAMBDA_3
        loss = loss_bpr + LAMBDA_1 * loss_cl + loss_pr + reg
        out[0:1, :] = jnp.full((1, 128), loss, _f32)
        out[1:2, :] = jnp.full((1, 128), loss_bpr, _f32)
        out[2:3, :] = jnp.full((1, 128), LAMBDA_1 * loss_cl, _f32)
        out[3:4, :] = jnp.full((1, 128), loss_pr, _f32)


def _final(zu_b, eu_b, zi_b, ei_b, ep_b, en_b, eut, eit, sqp, sqs, sumsa, prb):
    bb = pl.BlockSpec((B, D), lambda i: (0, 0))
    nb = pl.BlockSpec((CB, D), lambda i: (i, 0))
    s8 = pl.BlockSpec((8, 128), lambda i: (0, 0))
    return pl.pallas_call(
        _final_body,
        grid=(CGRID,),
        in_specs=[bb] * 6 + [nb, nb] + [s8] * 4,
        out_specs=s8,
        out_shape=jax.ShapeDtypeStruct((8, 128), _f32),
        scratch_shapes=[pltpu.VMEM((B, 1), _f32), pltpu.VMEM((B, 1), _f32)],
    )(zu_b, eu_b, zi_b, ei_b, ep_b, en_b, eut, eit, sqp, sqs, sumsa, prb)


def _spmm_pair(ti, tu, src3, dst3, w2, scaled):
    f = _spmm_scaled if scaled else _spmm_plain
    pu, pi = f(ti, tu, src3, dst3, w2)
    return _combine2(pu), _combine2(pi)


def kernel(uids, iids, pos, neg, edge_index, adj_vals, E_u_0, E_i_0, fuse_w,
           fuse_b, wv_param, W1, b1, W2, b2, a_u, a_i):
    src = edge_index[0].astype(_i32)
    dst = edge_index[1].astype(_i32)
    src3 = src.reshape(NW, NCH // G, G, C)
    dst3 = dst.reshape(NW, NCH // G, G, C)


    Eu0p = jnp.zeros((NPAD, D), _f32).at[:N_U].set(E_u_0)
    Ei0p = jnp.zeros((NPAD, D), _f32).at[:N_I].set(E_i_0)

    w2col = W2[:, 0]
    wau = jnp.zeros((D, 2 * D), _f32).at[:, :D].set(W1[:D]).at[:, D].set(a_u)
    wai = jnp.zeros((D, 2 * D), _f32).at[:, :D].set(W1[D:]).at[:, D].set(a_i)
    HuP, HiP, sqp = _prep(Eu0p, Ei0p, wau, wai)
    smalls = jnp.zeros((264, 128), _f32)
    smalls = smalls.at[:256].set(W1)
    smalls = smalls.at[256].set(a_u).at[257].set(a_i)
    smalls = smalls.at[258].set(b1).at[259].set(w2col)
    smalls = smalls.at[260, 0].set(b2[0])
    sqs = _smallred(smalls)

    Eu1, Ei1 = _spmm_pair(Ei0p, Eu0p, src3, dst3, adj_vals, True)
    Eu2, Ei2 = _spmm_pair(Ei1, Eu1, src3, dst3, adj_vals, True)
    E_u = _combine3(Eu0p, Eu1, Eu2)
    E_i = _combine3(Ei0p, Ei1, Ei2)

    bw = jnp.stack([b1, w2col])
    src3v = src.reshape(NW, NGRV, GV, CV)
    dst3v = dst.reshape(NW, NGRV, GV, CV)
    dot_gcn, dot_mlp, attsum = _views(E_u, E_i, HuP[:, :D], HiP[:, :D], bw,
                                      HuP[:N_U, D], HiP[:N_I, D], src3v, dst3v)

    def pad2d(x):
        return jnp.zeros((EP,), _f32).at[:E].set(x).reshape(EP // 128, 128)

    n0, n1, n2, n3, ggcn2, gsum2, sumsa = _edgea(
        b2, pad2d(dot_gcn), pad2d(dot_mlp), pad2d(attsum), pad2d(wv_param),
        pad2d(fuse_w), pad2d(fuse_b))
    aug2, prb = _edgeb(n0, n1, n2, n3, ggcn2, gsum2, pad2d(adj_vals), sumsa)
    aug_vals = aug2.reshape(EP)[:E]

    Zu1, Zi1 = _spmm_pair(Ei0p, Eu0p, src3, dst3, aug_vals, True)
    Zu2, Zi2 = _spmm_pair(Zi1, Zu1, src3, dst3, aug_vals, True)
    Z_u = _combine3(Eu0p, Zu1, Zu2)
    Z_i = _combine3(Ei0p, Zi1, Zi2)

    ub = uids.astype(_i32)
    ib = iids.astype(_i32)
    pb_ = pos.astype(_i32)
    nb_ = neg.astype(_i32)
    o_eu, o_zu, o_ei, o_zi, o_ep, o_en = _bgather(E_u, E_i, Z_u, Z_i,
                                                  ub, ib, pb_, nb_)
    fin = _final(o_zu, o_eu, o_zi, o_ei, o_ep, o_en, E_u, E_i,
                 sqp, sqs, sumsa, prb)
    return fin[0, 0], fin[1, 0], fin[2, 0], fin[3, 0]
